# baseline (speedup 1.0000x reference)
import jax
import jax.numpy as jnp
from jax import lax
from jax.experimental import pallas as pl
from jax.experimental.pallas import tpu as pltpu
from jax.experimental.pallas import tpu_sc as plsc

N_USER = 4000
N_ITEM = 6000
N = N_USER + N_ITEM
D = 128
Q = 96
E = 320000
N_LAYERS = 3

T = 16
C = 128
MB = 8
NCH = 160
NBLK = NCH // MB
PT = NCH * C
E_PAD = T * PT
NCHT = E_PAD // C
ACC_N = 10240
RPT = ACC_N // T
QL = Q // 16


def _spmm_body(x_hbm, src_hbm, dst_hbm, w_hbm, y_hbm,
               acc, srcb, dstb, wb, rows0, rows1, rows2, gsems, ssems):
    cid = lax.axis_index("c")
    sid = lax.axis_index("s")
    rows = (rows0, rows1, rows2)
    zv = jnp.zeros((16,), jnp.float32)

    def scale(rb, cj):
        @plsc.parallel_loop(0, C, unroll=2)
        def _s(e):
            wv = wb[cj, e]
            for c in range(QL):
                sl = pl.ds(c * 16, 16)
                rb[e, sl] = rb[e, sl] * wv

    for p in range(2):
        q = cid * 2 + p

        @plsc.parallel_loop(0, C)
        def _z(i):
            for c in range(QL):
                rows0[i, pl.ds(c * 16, 16)] = zv

        for k in range(RPT // C):
            pltpu.sync_copy(rows0, acc.at[pl.ds(sid * RPT + k * C, C)])
        plsc.subcore_barrier()

        def blk_body(blk, carry):
            r0 = sid * NCH + blk * MB
            pltpu.sync_copy(src_hbm.at[pl.ds(r0, MB)], srcb)
            pltpu.sync_copy(dst_hbm.at[pl.ds(r0, MB)], dstb)
            pltpu.sync_copy(w_hbm.at[pl.ds(r0, MB)], wb)

            off = jnp.full((16,), q * ACC_N, jnp.int32)
            for r in range(MB):
                for g in range(C // 16):
                    sl = pl.ds(g * 16, 16)
                    srcb[r, sl] = srcb[r, sl] + off

            pltpu.async_copy(x_hbm.at[srcb.at[0]], rows[0], gsems.at[0])
            for cj in range(MB):
                b = cj % 3
                if cj >= 2:
                    pltpu.make_async_copy(
                        rows[(cj + 1) % 3],
                        acc.at[dstb.at[cj - 2]],
                        ssems.at[(cj - 2) % 3]).wait()
                if cj + 1 < MB:
                    pltpu.async_copy(x_hbm.at[srcb.at[cj + 1]],
                                     rows[(cj + 1) % 3],
                                     gsems.at[(cj + 1) % 3])
                pltpu.make_async_copy(x_hbm.at[srcb.at[cj]], rows[b],
                                      gsems.at[b]).wait()
                scale(rows[b], cj)
                pltpu.async_copy(rows[b], acc.at[dstb.at[cj]],
                                 ssems.at[b], add=True)
            for cj in (MB - 2, MB - 1):
                pltpu.make_async_copy(rows[cj % 3], acc.at[dstb.at[cj]],
                                      ssems.at[cj % 3]).wait()
            return carry

        lax.fori_loop(0, NBLK, blk_body, 0)
        plsc.subcore_barrier()
        pltpu.sync_copy(acc.at[pl.ds(sid * RPT, RPT)],
                        y_hbm.at[pl.ds(q * ACC_N + sid * RPT, RPT)])
        plsc.subcore_barrier()


_spmm = pl.kernel(
    _spmm_body,
    out_type=jax.ShapeDtypeStruct((4 * ACC_N, Q), jnp.float32),
    mesh=plsc.VectorSubcoreMesh(core_axis_name="c", subcore_axis_name="s"),
    compiler_params=pltpu.CompilerParams(use_tc_tiling_on_sc=False),
    scratch_types=[
        pltpu.VMEM_SHARED((ACC_N, Q), jnp.float32),
        pltpu.VMEM((MB, C), jnp.int32),
        pltpu.VMEM((MB, C), jnp.int32),
        pltpu.VMEM((MB, C, 16), jnp.float32),
        pltpu.VMEM((C, Q), jnp.float32),
        pltpu.VMEM((C, Q), jnp.float32),
        pltpu.VMEM((C, Q), jnp.float32),
        pltpu.SemaphoreType.DMA((3,)),
        pltpu.SemaphoreType.DMA((3,)),
    ],
)


def _elem_body(y0, y1, y2, y3, mi, mt, se_in, si_in, st_in,
               x0_o, x1_o, x2_o, x3_o, se_o, si_o, st_o):
    b0 = y0[...]
    b1 = y1[...]
    b2 = y2[...]
    b3 = y3[...]
    ego = jnp.concatenate([b0, b1[:, :32]], axis=1)
    img = jnp.concatenate([b1[:, 32:], b2[:, :64]], axis=1)
    txt = jnp.concatenate([b2[:, 64:], b3], axis=1)

    li = jnp.where(img >= 0, img, 0.01 * img)
    di = li * mi[...]
    lt = jnp.where(txt >= 0, txt, 0.01 * txt)
    dt = lt * mt[...]

    nrm_i = jnp.sqrt(jnp.sum(di * di, axis=1, keepdims=True))
    ni = di / jnp.maximum(nrm_i, 1e-12)
    nrm_t = jnp.sqrt(jnp.sum(dt * dt, axis=1, keepdims=True))
    nt = dt / jnp.maximum(nrm_t, 1e-12)

    se_o[...] = se_in[...] + ego
    si_o[...] = si_in[...] + ni
    st_o[...] = st_in[...] + nt
    x0_o[...] = ego[:, :Q]
    x1_o[...] = jnp.concatenate([ego[:, Q:], di[:, :64]], axis=1)
    x2_o[...] = jnp.concatenate([di[:, 64:], dt[:, :32]], axis=1)
    x3_o[...] = dt[:, 32:]


_ELEM_R = 640


def _elem(y, mi, mt, se, si, st):
    g = ACC_N // _ELEM_R

    def bq(qq):
        return pl.BlockSpec((_ELEM_R, Q), lambda i, _q=qq: (_q * g + i, 0))

    bo = pl.BlockSpec((_ELEM_R, Q), lambda i: (i, 0))
    bd = pl.BlockSpec((_ELEM_R, D), lambda i: (i, 0))
    xt = jax.ShapeDtypeStruct((ACC_N, Q), jnp.float32)
    dt_ = jax.ShapeDtypeStruct((ACC_N, D), jnp.float32)
    return pl.pallas_call(
        _elem_body,
        grid=(g,),
        in_specs=[bq(0), bq(1), bq(2), bq(3), bd, bd, bd, bd, bd],
        out_specs=[bo, bo, bo, bo, bd, bd, bd],
        out_shape=[xt, xt, xt, xt, dt_, dt_, dt_],
    )(y, y, y, y, mi, mt, se, si, st)


def _user_body(se, o):
    o[...] = se[...] * (1.0 / 3.0)


def _item_body(se, si, st, f0, f1, f2, o):
    w0 = f0[0:1, 0:1]
    w1 = f1[0:1, 0:1]
    w2 = f2[0:1, 0:1]
    o[...] = (w0 * se[...] + w1 * si[...] + w2 * st[...]) * (1.0 / 3.0)


def _finalize(se, si, st, fw):
    f = [jnp.full((8, 128), fw[i], jnp.float32) for i in range(3)]
    bd = pl.BlockSpec((1000, D), lambda i: (i, 0))
    bf = pl.BlockSpec((8, 128), lambda i: (0, 0))
    user = pl.pallas_call(
        _user_body,
        grid=(4,),
        in_specs=[bd],
        out_specs=bd,
        out_shape=jax.ShapeDtypeStruct((N_USER, D), jnp.float32),
    )(se[:N_USER])
    item = pl.pallas_call(
        _item_body,
        grid=(6,),
        in_specs=[bd, bd, bd, bf, bf, bf],
        out_specs=bd,
        out_shape=jax.ShapeDtypeStruct((N_ITEM, D), jnp.float32),
    )(se[N_USER:N], si[N_USER:N], st[N_USER:N], f[0], f[1], f[2])
    return user, item


def _pad_rows(a):
    return jnp.concatenate(
        [a, jnp.zeros((ACC_N - N, a.shape[1]), a.dtype)], axis=0)


def kernel(user_emb, item_emb, image_emb, text_emb, fusion_weight,
           edge_index, edge_weight):
    src = edge_index[0].astype(jnp.int32)
    dst = edge_index[1].astype(jnp.int32)
    w = edge_weight.astype(jnp.float32)
    order = jnp.argsort(src)
    src = src[order]
    dst = dst[order]
    w = w[order]
    pad = E_PAD - E
    src = jnp.concatenate([src, jnp.zeros((pad,), jnp.int32)])
    dst = jnp.concatenate([dst, jnp.zeros((pad,), jnp.int32)])
    w = jnp.concatenate([w, jnp.zeros((pad,), jnp.float32)])
    src2d = src.reshape(NCHT, C)
    dst2d = dst.reshape(NCHT, C)
    w3d = jnp.broadcast_to(w[:, None], (E_PAD, 16)).reshape(NCHT, C, 16)

    ego = _pad_rows(jnp.concatenate([user_emb, item_emb], axis=0))
    img = _pad_rows(jnp.concatenate([user_emb, image_emb], axis=0))
    txt = _pad_rows(jnp.concatenate([user_emb, text_emb], axis=0))
    x = jnp.concatenate([
        ego[:, :Q],
        jnp.concatenate([ego[:, Q:], img[:, :64]], axis=1),
        jnp.concatenate([img[:, 64:], txt[:, :32]], axis=1),
        txt[:, 32:],
    ], axis=0)

    dk = jax.random.key(42)
    masks = []
    for k in range(N_LAYERS):
        mi = jax.random.bernoulli(jax.random.fold_in(dk, 2 * k), 0.5, (N, D))
        mt = jax.random.bernoulli(jax.random.fold_in(dk, 2 * k + 1), 0.5, (N, D))
        masks.append((_pad_rows(mi.astype(jnp.float32) * 2.0),
                      _pad_rows(mt.astype(jnp.float32) * 2.0)))

    se = jnp.zeros((ACC_N, D), jnp.float32)
    si = jnp.zeros((ACC_N, D), jnp.float32)
    st = jnp.zeros((ACC_N, D), jnp.float32)

    for k in range(N_LAYERS):
        y = _spmm(x, src2d, dst2d, w3d)
        x0, x1, x2, x3, se, si, st = _elem(y, masks[k][0], masks[k][1],
                                           se, si, st)
        x = jnp.concatenate([x0, x1, x2, x3], axis=0)

    return _finalize(se, si, st, fusion_weight)


# speedup vs baseline: 2.2471x; 2.2471x over previous
import jax
import jax.numpy as jnp
import numpy as np
from jax import lax
from jax.experimental import pallas as pl
from jax.experimental.pallas import tpu as pltpu
from jax.experimental.pallas import tpu_sc as plsc

N_USER = 4000
N_ITEM = 6000
N = N_USER + N_ITEM
D = 128
Q = 96
QW = Q // 2
E = 320000
N_LAYERS = 3

T = 16
C = 64
MB = 8
NCH = 320
NBLK = NCH // MB
PT = NCH * C
E_PAD = T * PT
NCHT = E_PAD // C
ACC_N = 10240
RPT = ACC_N // T
HI = np.uint32(0xFFFF0000)


def _spmm_body(x_hbm, src_hbm, dst_hbm, w_hbm, y_hbm,
               acc, xsp, srcb, dstb, wb, rows0, rows1, stg0, stg1,
               gsems, ssems):
    cid = lax.axis_index("c")
    sid = lax.axis_index("s")
    rows = (rows0, rows1)
    stg = (stg0, stg1)
    zv = jnp.zeros((16,), jnp.float32)

    def scale(rb, sb, cj):
        @plsc.parallel_loop(0, C, unroll=2)
        def _s(e):
            wv = wb[cj, e]
            for g in range(3):
                u = rb[e, pl.ds(g * 16, 16)]
                a = lax.bitcast_convert_type(u << 16, jnp.float32)
                b = lax.bitcast_convert_type(u & HI, jnp.float32)
                sb[e, pl.ds(g * 32, 16)] = a * wv
                sb[e, pl.ds(g * 32 + 16, 16)] = b * wv

    for p in range(2):
        q = cid * 2 + p

        pltpu.sync_copy(x_hbm.at[pl.ds(q * ACC_N + sid * RPT, RPT)],
                        xsp.at[pl.ds(sid * RPT, RPT)])

        @plsc.parallel_loop(0, C)
        def _z(i):
            for c in range(Q // 16):
                stg0[i, pl.ds(c * 16, 16)] = zv

        for k in range(RPT // C):
            pltpu.sync_copy(stg0, acc.at[pl.ds(sid * RPT + k * C, C)])
        plsc.subcore_barrier()

        def blk_body(blk, carry):
            r0 = sid * NBLK + blk
            pltpu.sync_copy(src_hbm.at[pl.ds(r0 * MB, MB)], srcb)
            pltpu.sync_copy(dst_hbm.at[pl.ds(r0 * MB, MB)], dstb)
            pltpu.sync_copy(w_hbm.at[pl.ds(r0 * MB, MB)], wb)

            pltpu.async_copy(xsp.at[srcb.at[0]], rows[0], gsems.at[0])
            for cj in range(MB):
                b = cj % 2
                if cj >= 2:
                    pltpu.make_async_copy(stg[b], acc.at[dstb.at[cj - 2]],
                                          ssems.at[b]).wait()
                if cj + 1 < MB:
                    pltpu.async_copy(xsp.at[srcb.at[cj + 1]],
                                     rows[1 - b], gsems.at[1 - b])
                pltpu.make_async_copy(xsp.at[srcb.at[cj]], rows[b],
                                      gsems.at[b]).wait()
                scale(rows[b], stg[b], cj)
                pltpu.async_copy(stg[b], acc.at[dstb.at[cj]],
                                 ssems.at[b], add=True)
            for cj in (MB - 2, MB - 1):
                pltpu.make_async_copy(stg[cj % 2], acc.at[dstb.at[cj]],
                                      ssems.at[cj % 2]).wait()
            return carry

        lax.fori_loop(0, NBLK, blk_body, 0)
        plsc.subcore_barrier()
        pltpu.sync_copy(acc.at[pl.ds(sid * RPT, RPT)],
                        y_hbm.at[pl.ds(q * ACC_N + sid * RPT, RPT)])
        plsc.subcore_barrier()


_spmm = pl.kernel(
    _spmm_body,
    out_type=jax.ShapeDtypeStruct((4 * ACC_N, Q), jnp.float32),
    mesh=plsc.VectorSubcoreMesh(core_axis_name="c", subcore_axis_name="s"),
    compiler_params=pltpu.CompilerParams(use_tc_tiling_on_sc=False),
    scratch_types=[
        pltpu.VMEM_SHARED((ACC_N, Q), jnp.float32),
        pltpu.VMEM_SHARED((ACC_N, QW), jnp.uint32),
        pltpu.VMEM((MB, C), jnp.int32),
        pltpu.VMEM((MB, C), jnp.int32),
        pltpu.VMEM((MB, C, 16), jnp.float32),
        pltpu.VMEM((C, QW), jnp.uint32),
        pltpu.VMEM((C, QW), jnp.uint32),
        pltpu.VMEM((C, Q), jnp.float32),
        pltpu.VMEM((C, Q), jnp.float32),
        pltpu.SemaphoreType.DMA((2,)),
        pltpu.SemaphoreType.DMA((2,)),
    ],
)


def _pack96(x):
    xb = lax.bitcast_convert_type(x.astype(jnp.bfloat16), jnp.uint16)
    xb = xb.astype(jnp.uint32)
    parts = []
    for g in range(3):
        lo = xb[:, g * 32:g * 32 + 16]
        hi = xb[:, g * 32 + 16:g * 32 + 32]
        parts.append(lo | (hi << 16))
    return jnp.concatenate(parts, axis=1)


def _elem_body(y0, y1, y2, y3, mi, mt, se_in, si_in, st_in,
               x0_o, x1_o, x2_o, x3_o, se_o, si_o, st_o):
    b0 = y0[...]
    b1 = y1[...]
    b2 = y2[...]
    b3 = y3[...]
    ego = jnp.concatenate([b0, b1[:, :32]], axis=1)
    img = jnp.concatenate([b1[:, 32:], b2[:, :64]], axis=1)
    txt = jnp.concatenate([b2[:, 64:], b3], axis=1)

    li = jnp.where(img >= 0, img, 0.01 * img)
    di = li * mi[...]
    lt = jnp.where(txt >= 0, txt, 0.01 * txt)
    dt = lt * mt[...]

    nrm_i = jnp.sqrt(jnp.sum(di * di, axis=1, keepdims=True))
    ni = di / jnp.maximum(nrm_i, 1e-12)
    nrm_t = jnp.sqrt(jnp.sum(dt * dt, axis=1, keepdims=True))
    nt = dt / jnp.maximum(nrm_t, 1e-12)

    se_o[...] = se_in[...] + ego
    si_o[...] = si_in[...] + ni
    st_o[...] = st_in[...] + nt
    x0_o[...] = _pack96(ego[:, :Q])
    x1_o[...] = _pack96(jnp.concatenate([ego[:, Q:], di[:, :64]], axis=1))
    x2_o[...] = _pack96(jnp.concatenate([di[:, 64:], dt[:, :32]], axis=1))
    x3_o[...] = _pack96(dt[:, 32:])


_ELEM_R = 640


def _elem(y, mi, mt, se, si, st):
    g = ACC_N // _ELEM_R

    def bq(qq):
        return pl.BlockSpec((_ELEM_R, Q), lambda i, _q=qq: (_q * g + i, 0))

    bx = pl.BlockSpec((_ELEM_R, QW), lambda i: (i, 0))
    bd = pl.BlockSpec((_ELEM_R, D), lambda i: (i, 0))
    xt = jax.ShapeDtypeStruct((ACC_N, QW), jnp.uint32)
    dt_ = jax.ShapeDtypeStruct((ACC_N, D), jnp.float32)
    return pl.pallas_call(
        _elem_body,
        grid=(g,),
        in_specs=[bq(0), bq(1), bq(2), bq(3), bd, bd, bd, bd, bd],
        out_specs=[bx, bx, bx, bx, bd, bd, bd],
        out_shape=[xt, xt, xt, xt, dt_, dt_, dt_],
    )(y, y, y, y, mi, mt, se, si, st)


def _user_body(se, o):
    o[...] = se[...] * (1.0 / 3.0)


def _item_body(se, si, st, f0, f1, f2, o):
    w0 = f0[0:1, 0:1]
    w1 = f1[0:1, 0:1]
    w2 = f2[0:1, 0:1]
    o[...] = (w0 * se[...] + w1 * si[...] + w2 * st[...]) * (1.0 / 3.0)


def _finalize(se, si, st, fw):
    f = [jnp.full((8, 128), fw[i], jnp.float32) for i in range(3)]
    bd = pl.BlockSpec((1000, D), lambda i: (i, 0))
    bf = pl.BlockSpec((8, 128), lambda i: (0, 0))
    user = pl.pallas_call(
        _user_body,
        grid=(4,),
        in_specs=[bd],
        out_specs=bd,
        out_shape=jax.ShapeDtypeStruct((N_USER, D), jnp.float32),
    )(se[:N_USER])
    item = pl.pallas_call(
        _item_body,
        grid=(6,),
        in_specs=[bd, bd, bd, bf, bf, bf],
        out_specs=bd,
        out_shape=jax.ShapeDtypeStruct((N_ITEM, D), jnp.float32),
    )(se[N_USER:N], si[N_USER:N], st[N_USER:N], f[0], f[1], f[2])
    return user, item


def _pad_rows(a):
    return jnp.concatenate(
        [a, jnp.zeros((ACC_N - N, a.shape[1]), a.dtype)], axis=0)


def kernel(user_emb, item_emb, image_emb, text_emb, fusion_weight,
           edge_index, edge_weight):
    src = edge_index[0].astype(jnp.int32)
    dst = edge_index[1].astype(jnp.int32)
    w = edge_weight.astype(jnp.float32)
    pad = E_PAD - E
    src = jnp.concatenate([src, jnp.zeros((pad,), jnp.int32)])
    dst = jnp.concatenate([dst, jnp.zeros((pad,), jnp.int32)])
    w = jnp.concatenate([w, jnp.zeros((pad,), jnp.float32)])
    src2d = src.reshape(NCHT, C)
    dst2d = dst.reshape(NCHT, C)
    w3d = jnp.broadcast_to(w[:, None], (E_PAD, 16)).reshape(NCHT, C, 16)

    ego = _pad_rows(jnp.concatenate([user_emb, item_emb], axis=0))
    img = _pad_rows(jnp.concatenate([user_emb, image_emb], axis=0))
    txt = _pad_rows(jnp.concatenate([user_emb, text_emb], axis=0))
    x = jnp.concatenate([
        _pack96(ego[:, :Q]),
        _pack96(jnp.concatenate([ego[:, Q:], img[:, :64]], axis=1)),
        _pack96(jnp.concatenate([img[:, 64:], txt[:, :32]], axis=1)),
        _pack96(txt[:, 32:]),
    ], axis=0)

    dk = jax.random.key(42)
    masks = []
    for k in range(N_LAYERS):
        mi = jax.random.bernoulli(jax.random.fold_in(dk, 2 * k), 0.5, (N, D))
        mt = jax.random.bernoulli(jax.random.fold_in(dk, 2 * k + 1), 0.5, (N, D))
        masks.append((_pad_rows(mi.astype(jnp.float32) * 2.0),
                      _pad_rows(mt.astype(jnp.float32) * 2.0)))

    se = jnp.zeros((ACC_N, D), jnp.float32)
    si = jnp.zeros((ACC_N, D), jnp.float32)
    st = jnp.zeros((ACC_N, D), jnp.float32)

    for k in range(N_LAYERS):
        y = _spmm(x, src2d, dst2d, w3d)
        x0, x1, x2, x3, se, si, st = _elem(y, masks[k][0], masks[k][1],
                                           se, si, st)
        x = jnp.concatenate([x0, x1, x2, x3], axis=0)

    return _finalize(se, si, st, fusion_weight)


# speedup vs baseline: 2.4874x; 1.1069x over previous
import jax
import jax.numpy as jnp
import numpy as np
from jax import lax
from jax.experimental import pallas as pl
from jax.experimental.pallas import tpu as pltpu
from jax.experimental.pallas import tpu_sc as plsc

N_USER = 4000
N_ITEM = 6000
N = N_USER + N_ITEM
D = 128
Q = 96
QW = Q // 2
E = 320000
N_LAYERS = 3

T = 16
C = 64
MB = 8
NCH = 320
NBLK = NCH // MB
PT = NCH * C
E_PAD = T * PT
NCHT = E_PAD // C
ACC_N = 10240
RPT = ACC_N // T
HI = np.uint32(0xFFFF0000)


def _spmm_body(x_hbm, src_hbm, dst_hbm, w_hbm, y_hbm,
               acc, xsp, srcb0, srcb1, dstb0, dstb1, wb0, wb1,
               rows0, rows1, stg0, stg1, gsems, ssems, msems):
    cid = lax.axis_index("c")
    sid = lax.axis_index("s")
    rows = (rows0, rows1)
    stg = (stg0, stg1)
    meta = ((srcb0, dstb0, wb0), (srcb1, dstb1, wb1))
    zv = jnp.zeros((16,), jnp.float32)

    def meta_start(b, slot):
        srcb, dstb, wb = meta[slot]
        r0 = (sid * NBLK + b) * MB
        sem = msems.at[slot]
        pltpu.async_copy(src_hbm.at[pl.ds(r0, MB)], srcb, sem)
        pltpu.async_copy(dst_hbm.at[pl.ds(r0, MB)], dstb, sem)
        pltpu.async_copy(w_hbm.at[pl.ds(r0, MB)], wb, sem)

    def meta_wait(b, slot):
        srcb, dstb, wb = meta[slot]
        r0 = (sid * NBLK + b) * MB
        sem = msems.at[slot]
        pltpu.make_async_copy(src_hbm.at[pl.ds(r0, MB)], srcb, sem).wait()
        pltpu.make_async_copy(dst_hbm.at[pl.ds(r0, MB)], dstb, sem).wait()
        pltpu.make_async_copy(w_hbm.at[pl.ds(r0, MB)], wb, sem).wait()

    def scale(rb, sb, wb, cj):
        @plsc.parallel_loop(0, C, unroll=2)
        def _s(e):
            wv = wb[cj, e]
            for g in range(3):
                u = rb[e, pl.ds(g * 16, 16)]
                a = lax.bitcast_convert_type(u << 16, jnp.float32)
                b = lax.bitcast_convert_type(u & HI, jnp.float32)
                sb[e, pl.ds(g * 32, 16)] = a * wv
                sb[e, pl.ds(g * 32 + 16, 16)] = b * wv

    for p in range(2):
        q = cid * 2 + p

        pltpu.sync_copy(x_hbm.at[pl.ds(q * ACC_N + sid * RPT, RPT)],
                        xsp.at[pl.ds(sid * RPT, RPT)])

        @plsc.parallel_loop(0, C)
        def _z(i):
            for c in range(Q // 16):
                stg0[i, pl.ds(c * 16, 16)] = zv

        for k in range(RPT // C):
            pltpu.sync_copy(stg0, acc.at[pl.ds(sid * RPT + k * C, C)])
        plsc.subcore_barrier()

        def do_block(blk, slot):
            srcb, dstb, wb = meta[slot]
            pltpu.async_copy(xsp.at[srcb.at[0]], rows[0], gsems.at[0])
            for cj in range(MB):
                b = cj % 2
                if cj >= 2:
                    pltpu.make_async_copy(stg[b], acc.at[dstb.at[cj - 2]],
                                          ssems.at[b]).wait()
                if cj + 1 < MB:
                    pltpu.async_copy(xsp.at[srcb.at[cj + 1]],
                                     rows[1 - b], gsems.at[1 - b])
                pltpu.make_async_copy(xsp.at[srcb.at[cj]], rows[b],
                                      gsems.at[b]).wait()
                scale(rows[b], stg[b], wb, cj)
                pltpu.async_copy(stg[b], acc.at[dstb.at[cj]],
                                 ssems.at[b], add=True)
            for cj in (MB - 2, MB - 1):
                pltpu.make_async_copy(stg[cj % 2], acc.at[dstb.at[cj]],
                                      ssems.at[cj % 2]).wait()

        meta_start(0, 0)

        def blk2_body(t, carry):
            b0 = 2 * t
            meta_wait(b0, 0)
            meta_start(b0 + 1, 1)
            do_block(b0, 0)
            meta_wait(b0 + 1, 1)

            @pl.when(t + 1 < NBLK // 2)
            def _():
                meta_start(b0 + 2, 0)

            do_block(b0 + 1, 1)
            return carry

        lax.fori_loop(0, NBLK // 2, blk2_body, 0)
        plsc.subcore_barrier()
        pltpu.sync_copy(acc.at[pl.ds(sid * RPT, RPT)],
                        y_hbm.at[pl.ds(q * ACC_N + sid * RPT, RPT)])
        plsc.subcore_barrier()


_spmm = pl.kernel(
    _spmm_body,
    out_type=jax.ShapeDtypeStruct((4 * ACC_N, Q), jnp.float32),
    mesh=plsc.VectorSubcoreMesh(core_axis_name="c", subcore_axis_name="s"),
    compiler_params=pltpu.CompilerParams(use_tc_tiling_on_sc=False),
    scratch_types=[
        pltpu.VMEM_SHARED((ACC_N, Q), jnp.float32),
        pltpu.VMEM_SHARED((ACC_N, QW), jnp.uint32),
        pltpu.VMEM((MB, C), jnp.int32),
        pltpu.VMEM((MB, C), jnp.int32),
        pltpu.VMEM((MB, C), jnp.int32),
        pltpu.VMEM((MB, C), jnp.int32),
        pltpu.VMEM((MB, C, 16), jnp.float32),
        pltpu.VMEM((MB, C, 16), jnp.float32),
        pltpu.VMEM((C, QW), jnp.uint32),
        pltpu.VMEM((C, QW), jnp.uint32),
        pltpu.VMEM((C, Q), jnp.float32),
        pltpu.VMEM((C, Q), jnp.float32),
        pltpu.SemaphoreType.DMA((2,)),
        pltpu.SemaphoreType.DMA((2,)),
        pltpu.SemaphoreType.DMA((2,)),
    ],
)


def _pack96(x):
    xb = lax.bitcast_convert_type(x.astype(jnp.bfloat16), jnp.uint16)
    xb = xb.astype(jnp.uint32)
    parts = []
    for g in range(3):
        lo = xb[:, g * 32:g * 32 + 16]
        hi = xb[:, g * 32 + 16:g * 32 + 32]
        parts.append(lo | (hi << 16))
    return jnp.concatenate(parts, axis=1)


def _elem_body(y0, y1, y2, y3, mi, mt, se_in, si_in, st_in,
               x0_o, x1_o, x2_o, x3_o, se_o, si_o, st_o):
    b0 = y0[...]
    b1 = y1[...]
    b2 = y2[...]
    b3 = y3[...]
    ego = jnp.concatenate([b0, b1[:, :32]], axis=1)
    img = jnp.concatenate([b1[:, 32:], b2[:, :64]], axis=1)
    txt = jnp.concatenate([b2[:, 64:], b3], axis=1)

    li = jnp.where(img >= 0, img, 0.01 * img)
    di = li * mi[...]
    lt = jnp.where(txt >= 0, txt, 0.01 * txt)
    dt = lt * mt[...]

    nrm_i = jnp.sqrt(jnp.sum(di * di, axis=1, keepdims=True))
    ni = di / jnp.maximum(nrm_i, 1e-12)
    nrm_t = jnp.sqrt(jnp.sum(dt * dt, axis=1, keepdims=True))
    nt = dt / jnp.maximum(nrm_t, 1e-12)

    se_o[...] = se_in[...] + ego
    si_o[...] = si_in[...] + ni
    st_o[...] = st_in[...] + nt
    x0_o[...] = _pack96(ego[:, :Q])
    x1_o[...] = _pack96(jnp.concatenate([ego[:, Q:], di[:, :64]], axis=1))
    x2_o[...] = _pack96(jnp.concatenate([di[:, 64:], dt[:, :32]], axis=1))
    x3_o[...] = _pack96(dt[:, 32:])


_ELEM_R = 640


def _elem(y, mi, mt, se, si, st):
    g = ACC_N // _ELEM_R

    def bq(qq):
        return pl.BlockSpec((_ELEM_R, Q), lambda i, _q=qq: (_q * g + i, 0))

    bx = pl.BlockSpec((_ELEM_R, QW), lambda i: (i, 0))
    bd = pl.BlockSpec((_ELEM_R, D), lambda i: (i, 0))
    xt = jax.ShapeDtypeStruct((ACC_N, QW), jnp.uint32)
    dt_ = jax.ShapeDtypeStruct((ACC_N, D), jnp.float32)
    return pl.pallas_call(
        _elem_body,
        grid=(g,),
        in_specs=[bq(0), bq(1), bq(2), bq(3), bd, bd, bd, bd, bd],
        out_specs=[bx, bx, bx, bx, bd, bd, bd],
        out_shape=[xt, xt, xt, xt, dt_, dt_, dt_],
    )(y, y, y, y, mi, mt, se, si, st)


def _user_body(se, o):
    o[...] = se[...] * (1.0 / 3.0)


def _item_body(se, si, st, f0, f1, f2, o):
    w0 = f0[0:1, 0:1]
    w1 = f1[0:1, 0:1]
    w2 = f2[0:1, 0:1]
    o[...] = (w0 * se[...] + w1 * si[...] + w2 * st[...]) * (1.0 / 3.0)


def _finalize(se, si, st, fw):
    f = [jnp.full((8, 128), fw[i], jnp.float32) for i in range(3)]
    bd = pl.BlockSpec((1000, D), lambda i: (i, 0))
    bf = pl.BlockSpec((8, 128), lambda i: (0, 0))
    user = pl.pallas_call(
        _user_body,
        grid=(4,),
        in_specs=[bd],
        out_specs=bd,
        out_shape=jax.ShapeDtypeStruct((N_USER, D), jnp.float32),
    )(se[:N_USER])
    item = pl.pallas_call(
        _item_body,
        grid=(6,),
        in_specs=[bd, bd, bd, bf, bf, bf],
        out_specs=bd,
        out_shape=jax.ShapeDtypeStruct((N_ITEM, D), jnp.float32),
    )(se[N_USER:N], si[N_USER:N], st[N_USER:N], f[0], f[1], f[2])
    return user, item


def _pad_rows(a):
    return jnp.concatenate(
        [a, jnp.zeros((ACC_N - N, a.shape[1]), a.dtype)], axis=0)


def kernel(user_emb, item_emb, image_emb, text_emb, fusion_weight,
           edge_index, edge_weight):
    src = edge_index[0].astype(jnp.int32)
    dst = edge_index[1].astype(jnp.int32)
    w = edge_weight.astype(jnp.float32)
    pad = E_PAD - E
    src = jnp.concatenate([src, jnp.zeros((pad,), jnp.int32)])
    dst = jnp.concatenate([dst, jnp.zeros((pad,), jnp.int32)])
    w = jnp.concatenate([w, jnp.zeros((pad,), jnp.float32)])
    src2d = src.reshape(NCHT, C)
    dst2d = dst.reshape(NCHT, C)
    w3d = jnp.broadcast_to(w[:, None], (E_PAD, 16)).reshape(NCHT, C, 16)

    ego = _pad_rows(jnp.concatenate([user_emb, item_emb], axis=0))
    img = _pad_rows(jnp.concatenate([user_emb, image_emb], axis=0))
    txt = _pad_rows(jnp.concatenate([user_emb, text_emb], axis=0))
    x = jnp.concatenate([
        _pack96(ego[:, :Q]),
        _pack96(jnp.concatenate([ego[:, Q:], img[:, :64]], axis=1)),
        _pack96(jnp.concatenate([img[:, 64:], txt[:, :32]], axis=1)),
        _pack96(txt[:, 32:]),
    ], axis=0)

    dk = jax.random.key(42)
    masks = []
    for k in range(N_LAYERS):
        mi = jax.random.bernoulli(jax.random.fold_in(dk, 2 * k), 0.5, (N, D))
        mt = jax.random.bernoulli(jax.random.fold_in(dk, 2 * k + 1), 0.5, (N, D))
        masks.append((_pad_rows(mi.astype(jnp.float32) * 2.0),
                      _pad_rows(mt.astype(jnp.float32) * 2.0)))

    se = jnp.zeros((ACC_N, D), jnp.float32)
    si = jnp.zeros((ACC_N, D), jnp.float32)
    st = jnp.zeros((ACC_N, D), jnp.float32)

    for k in range(N_LAYERS):
        y = _spmm(x, src2d, dst2d, w3d)
        x0, x1, x2, x3, se, si, st = _elem(y, masks[k][0], masks[k][1],
                                           se, si, st)
        x = jnp.concatenate([x0, x1, x2, x3], axis=0)

    return _finalize(se, si, st, fusion_weight)
